# baseline (speedup 1.0000x reference)
import jax
import jax.numpy as jnp
from jax import lax
from jax.experimental import pallas as pl
from jax.experimental.pallas import tpu as pltpu
from jax.experimental.pallas import tpu_sc as plsc

N_ATOMS = 16384
N_SYS = 16
D = 512
N_SPECIES = 100

NTILES = 16
CHUNK = 16
NGROUP = 7
SPECIES_PAD = NGROUP * CHUNK
DHALF = D // 2
APT = N_ATOMS // NTILES
NCHUNK = APT // CHUNK

_MAGIC = 0x5F3759DF


def _sc_body(pk_hbm, e_hbm, wb_hbm, bvec_hbm, out_hbm,
             pk_loc, e_loc, w_loc, vchunk, v01_loc, v_loc,
             acc_loc, red_loc, bvec_loc, tot_loc,
             shared_v01, shared_acc, sem_pk, sem_ew, sem_b):
    wid = lax.axis_index("s")

    cp_pk = pltpu.async_copy(pk_hbm.at[:, pl.ds(wid * APT, APT)], pk_loc, sem_pk)

    @pl.when(wid == 0)
    def _():
        pltpu.async_copy(bvec_hbm, bvec_loc, sem_b)

    @pl.when(wid < 2 * NGROUP)
    def _():
        g = wid // 2
        h = wid % 2
        cp_e = pltpu.async_copy(e_hbm.at[wid], e_loc, sem_ew)
        cp_w = pltpu.async_copy(
            wb_hbm.at[pl.ds(h * DHALF, DHALF)], w_loc, sem_ew)
        cp_e.wait()
        cp_w.wait()
        vaccs = [jnp.zeros((CHUNK,), jnp.float32) for _ in range(4)]
        for d in range(DHALF):
            vaccs[d % 4] = vaccs[d % 4] + w_loc[d] * e_loc[d]
        vchunk[...] = (vaccs[0] + vaccs[1]) + (vaccs[2] + vaccs[3])
        pltpu.sync_copy(
            vchunk, shared_v01.at[pl.ds(h * SPECIES_PAD + g * CHUNK, CHUNK)])

    plsc.subcore_barrier()
    pltpu.sync_copy(shared_v01, v01_loc)
    for c in range(NGROUP):
        sl = pl.ds(c * CHUNK, CHUNK)
        v_loc[sl] = v01_loc[sl] + v01_loc[pl.ds(SPECIES_PAD + c * CHUNK, CHUNK)]

    cp_pk.wait()

    acc_loc[...] = jnp.zeros((N_SYS,), jnp.float32)
    seg0_vec = plsc.bitcast(pk_loc[4, pl.ds(0, CHUNK)], jnp.int32)

    def chunk_body(c, carry):
        vacc, cur = carry
        base = c * CHUNK
        sl = pl.ds(base, CHUNK)
        x = pk_loc[0, sl]
        y = pk_loc[1, sl]
        z = pk_loc[2, sl]
        a = plsc.bitcast(pk_loc[3, sl], jnp.int32)
        seg = plsc.bitcast(pk_loc[4, sl], jnp.int32)
        rr = x * x + y * y + z * z
        w = plsc.bitcast(_MAGIC - (plsc.bitcast(rr, jnp.int32) >> 1), jnp.float32)
        half = rr * 0.5
        for _ in range(2):
            w = w * (1.5 - half * w * w)
        r = rr * w
        va = plsc.load_gather(v_loc, [a])
        val = r * va
        uniform = seg[CHUNK - 1] == cur

        def fast():
            return vacc + val, cur

        def slow():
            plsc.addupdate_scatter(acc_loc, [jnp.full((CHUNK,), cur, jnp.int32)],
                                   vacc)
            plsc.addupdate_scatter(acc_loc, [seg], val)
            return jnp.zeros((CHUNK,), jnp.float32), seg[CHUNK - 1]

        return lax.cond(uniform, fast, slow)

    vacc, cur = lax.fori_loop(
        0, NCHUNK, chunk_body,
        (jnp.zeros((CHUNK,), jnp.float32), seg0_vec[0]))
    plsc.addupdate_scatter(acc_loc, [jnp.full((CHUNK,), cur, jnp.int32)], vacc)

    pltpu.sync_copy(acc_loc, shared_acc.at[pl.ds(wid * N_SYS, N_SYS)])
    plsc.subcore_barrier()

    @pl.when(wid == 0)
    def _():
        pltpu.sync_copy(shared_acc, red_loc)
        pltpu.make_async_copy(bvec_hbm, bvec_loc, sem_b).wait()
        tot = bvec_loc[...]
        for i in range(NTILES):
            tot = tot + red_loc[pl.ds(i * N_SYS, N_SYS)]
        tot_loc[...] = tot
        pltpu.sync_copy(tot_loc, out_hbm)


_sc_kernel = pl.kernel(
    _sc_body,
    out_type=jax.ShapeDtypeStruct((N_SYS,), jnp.float32),
    mesh=plsc.VectorSubcoreMesh(core_axis_name="c", subcore_axis_name="s",
                                num_cores=1, num_subcores=NTILES),
    compiler_params=pltpu.CompilerParams(needs_layout_passes=False),
    scratch_types=[
        pltpu.VMEM((5, APT), jnp.float32),
        pltpu.VMEM((DHALF, CHUNK), jnp.float32),
        pltpu.VMEM((DHALF, CHUNK), jnp.float32),
        pltpu.VMEM((CHUNK,), jnp.float32),
        pltpu.VMEM((2 * SPECIES_PAD,), jnp.float32),
        pltpu.VMEM((SPECIES_PAD,), jnp.float32),
        pltpu.VMEM((N_SYS,), jnp.float32),
        pltpu.VMEM((NTILES * N_SYS,), jnp.float32),
        pltpu.VMEM((N_SYS,), jnp.float32),
        pltpu.VMEM((N_SYS,), jnp.float32),
        pltpu.VMEM_SHARED((2 * SPECIES_PAD,), jnp.float32),
        pltpu.VMEM_SHARED((NTILES * N_SYS,), jnp.float32),
        pltpu.SemaphoreType.DMA,
        pltpu.SemaphoreType.DMA,
        pltpu.SemaphoreType.DMA,
    ],
)


def kernel(atomic_numbers, pos, batch, species_embed, W, b):
    posf = pos.astype(jnp.float32)
    an_f = lax.bitcast_convert_type(atomic_numbers.astype(jnp.int32), jnp.float32)
    bt_f = lax.bitcast_convert_type(batch.astype(jnp.int32), jnp.float32)
    packed = jnp.stack([posf[:, 0], posf[:, 1], posf[:, 2], an_f, bt_f])
    epad = jnp.pad(species_embed.astype(jnp.float32),
                   ((0, SPECIES_PAD - N_SPECIES), (0, 0)))
    eslab = (epad.reshape(NGROUP, CHUNK, 2, DHALF)
                 .transpose(0, 2, 3, 1)
                 .reshape(2 * NGROUP, DHALF, CHUNK))
    wb = jnp.broadcast_to(W.reshape(-1, 1).astype(jnp.float32), (D, CHUNK))
    bvec = jnp.broadcast_to(b.astype(jnp.float32), (N_SYS,))
    y = _sc_kernel(packed, eslab, wb, bvec)
    return y.reshape(N_SYS, 1)


# speedup vs baseline: 1.1140x; 1.1140x over previous
import jax
import jax.numpy as jnp
from jax import lax
from jax.experimental import pallas as pl
from jax.experimental.pallas import tpu as pltpu
from jax.experimental.pallas import tpu_sc as plsc

N_ATOMS = 16384
N_SYS = 16
D = 512
N_SPECIES = 100

NTILES = 16
CHUNK = 16
NGROUP = 7
SPECIES_PAD = NGROUP * CHUNK
DHALF = D // 2
APT = N_ATOMS // NTILES
NCHUNK = APT // CHUNK

_MAGIC = 0x5F3759DF


def _sc_body(pk_hbm, e_hbm, w_hbm, bvec_hbm, out_hbm,
             pk_loc, e_loc, w_loc, vchunk, v01_loc, v_loc,
             acc_loc, red_loc, bvec_loc, tot_loc,
             shared_v01, shared_acc, sem_pk, sem_ew, sem_b):
    wid = lax.axis_index("s")

    cp_pk = pltpu.async_copy(pk_hbm.at[:, pl.ds(wid * APT, APT)], pk_loc, sem_pk)

    @pl.when(wid == 0)
    def _():
        pltpu.async_copy(bvec_hbm, bvec_loc, sem_b)

    @pl.when(wid < 2 * NGROUP)
    def _():
        g = wid // 2
        h = wid % 2
        cp_e = pltpu.async_copy(e_hbm.at[wid], e_loc, sem_ew)
        cp_w = pltpu.async_copy(w_hbm.at[pl.ds(h * DHALF, DHALF)], w_loc, sem_ew)
        cp_e.wait()
        cp_w.wait()
        vreg = jnp.zeros((CHUNK,), jnp.float32)
        for d0 in range(0, DHALF, CHUNK):
            wv = w_loc[pl.ds(d0, CHUNK)]
            for j in range(CHUNK):
                vreg = vreg + wv[j] * e_loc[d0 + j]
        vchunk[...] = vreg
        pltpu.sync_copy(
            vchunk, shared_v01.at[pl.ds(h * SPECIES_PAD + g * CHUNK, CHUNK)])

    plsc.subcore_barrier()
    pltpu.sync_copy(shared_v01, v01_loc)
    for c in range(NGROUP):
        sl = pl.ds(c * CHUNK, CHUNK)
        v_loc[sl] = v01_loc[sl] + v01_loc[pl.ds(SPECIES_PAD + c * CHUNK, CHUNK)]

    cp_pk.wait()

    acc_loc[...] = jnp.zeros((N_SYS,), jnp.float32)
    seg0_vec = plsc.bitcast(pk_loc[4, pl.ds(0, CHUNK)], jnp.int32)

    def chunk_body(c, carry):
        vacc, cur = carry
        base = c * CHUNK
        sl = pl.ds(base, CHUNK)
        x = pk_loc[0, sl]
        y = pk_loc[1, sl]
        z = pk_loc[2, sl]
        a = plsc.bitcast(pk_loc[3, sl], jnp.int32)
        seg = plsc.bitcast(pk_loc[4, sl], jnp.int32)
        rr = x * x + y * y + z * z
        w = plsc.bitcast(_MAGIC - (plsc.bitcast(rr, jnp.int32) >> 1), jnp.float32)
        half = rr * 0.5
        for _ in range(2):
            w = w * (1.5 - half * w * w)
        r = rr * w
        va = plsc.load_gather(v_loc, [a])
        val = r * va
        uniform = seg[CHUNK - 1] == cur

        def fast():
            return vacc + val, cur

        def slow():
            plsc.addupdate_scatter(acc_loc, [jnp.full((CHUNK,), cur, jnp.int32)],
                                   vacc)
            plsc.addupdate_scatter(acc_loc, [seg], val)
            return jnp.zeros((CHUNK,), jnp.float32), seg[CHUNK - 1]

        return lax.cond(uniform, fast, slow)

    vacc, cur = lax.fori_loop(
        0, NCHUNK, chunk_body,
        (jnp.zeros((CHUNK,), jnp.float32), seg0_vec[0]))
    plsc.addupdate_scatter(acc_loc, [jnp.full((CHUNK,), cur, jnp.int32)], vacc)

    pltpu.sync_copy(acc_loc, shared_acc.at[pl.ds(wid * N_SYS, N_SYS)])
    plsc.subcore_barrier()

    @pl.when(wid == 0)
    def _():
        pltpu.sync_copy(shared_acc, red_loc)
        pltpu.make_async_copy(bvec_hbm, bvec_loc, sem_b).wait()
        tot = bvec_loc[...]
        for i in range(NTILES):
            tot = tot + red_loc[pl.ds(i * N_SYS, N_SYS)]
        tot_loc[...] = tot
        pltpu.sync_copy(tot_loc, out_hbm)


_sc_kernel = pl.kernel(
    _sc_body,
    out_type=jax.ShapeDtypeStruct((N_SYS,), jnp.float32),
    mesh=plsc.VectorSubcoreMesh(core_axis_name="c", subcore_axis_name="s",
                                num_cores=1, num_subcores=NTILES),
    compiler_params=pltpu.CompilerParams(needs_layout_passes=False),
    scratch_types=[
        pltpu.VMEM((5, APT), jnp.float32),
        pltpu.VMEM((DHALF, CHUNK), jnp.float32),
        pltpu.VMEM((DHALF,), jnp.float32),
        pltpu.VMEM((CHUNK,), jnp.float32),
        pltpu.VMEM((2 * SPECIES_PAD,), jnp.float32),
        pltpu.VMEM((SPECIES_PAD,), jnp.float32),
        pltpu.VMEM((N_SYS,), jnp.float32),
        pltpu.VMEM((NTILES * N_SYS,), jnp.float32),
        pltpu.VMEM((N_SYS,), jnp.float32),
        pltpu.VMEM((N_SYS,), jnp.float32),
        pltpu.VMEM_SHARED((2 * SPECIES_PAD,), jnp.float32),
        pltpu.VMEM_SHARED((NTILES * N_SYS,), jnp.float32),
        pltpu.SemaphoreType.DMA,
        pltpu.SemaphoreType.DMA,
        pltpu.SemaphoreType.DMA,
    ],
)


def kernel(atomic_numbers, pos, batch, species_embed, W, b):
    posf = pos.astype(jnp.float32)
    an_f = lax.bitcast_convert_type(atomic_numbers.astype(jnp.int32), jnp.float32)
    bt_f = lax.bitcast_convert_type(batch.astype(jnp.int32), jnp.float32)
    packed = jnp.stack([posf[:, 0], posf[:, 1], posf[:, 2], an_f, bt_f])
    epad = jnp.pad(species_embed.astype(jnp.float32),
                   ((0, SPECIES_PAD - N_SPECIES), (0, 0)))
    eslab = (epad.reshape(NGROUP, CHUNK, 2, DHALF)
                 .transpose(0, 2, 3, 1)
                 .reshape(2 * NGROUP, DHALF, CHUNK))
    wf = W.reshape(-1).astype(jnp.float32)
    bvec = jnp.broadcast_to(b.astype(jnp.float32), (N_SYS,))
    y = _sc_kernel(packed, eslab, wf, bvec)
    return y.reshape(N_SYS, 1)


# speedup vs baseline: 1.1453x; 1.0281x over previous
import jax
import jax.numpy as jnp
from jax import lax
from jax.experimental import pallas as pl
from jax.experimental.pallas import tpu as pltpu
from jax.experimental.pallas import tpu_sc as plsc

N_ATOMS = 16384
N_SYS = 16
D = 512
N_SPECIES = 100

NTILES = 16
CHUNK = 16
NGROUP = 7
SPECIES_PAD = NGROUP * CHUNK
DHALF = D // 2
APT = N_ATOMS // NTILES
NCHUNK = APT // CHUNK
BLK = 4

_MAGIC = 0x5F3759DF


def _sc_body(pk_hbm, e_hbm, w_hbm, bvec_hbm, out_hbm,
             pk_loc, e_loc, w_loc, vchunk, v01_loc, v_loc,
             acc_loc, red_loc, bvec_loc, tot_loc,
             shared_v01, shared_acc, sem_pk, sem_ew, sem_b):
    wid = lax.axis_index("s")

    cp_pk = pltpu.async_copy(pk_hbm.at[:, pl.ds(wid * APT, APT)], pk_loc, sem_pk)

    @pl.when(wid == 0)
    def _():
        pltpu.async_copy(bvec_hbm, bvec_loc, sem_b)

    @pl.when(wid < 2 * NGROUP)
    def _():
        g = wid // 2
        h = wid % 2
        cp_e = pltpu.async_copy(e_hbm.at[wid], e_loc, sem_ew)
        cp_w = pltpu.async_copy(w_hbm.at[pl.ds(h * DHALF, DHALF)], w_loc, sem_ew)
        cp_e.wait()
        cp_w.wait()
        vreg = jnp.zeros((CHUNK,), jnp.float32)
        for d0 in range(0, DHALF, CHUNK):
            wv = w_loc[pl.ds(d0, CHUNK)]
            for j in range(CHUNK):
                vreg = vreg + wv[j] * e_loc[d0 + j]
        vchunk[...] = vreg
        pltpu.sync_copy(
            vchunk, shared_v01.at[pl.ds(h * SPECIES_PAD + g * CHUNK, CHUNK)])

    plsc.subcore_barrier()
    pltpu.sync_copy(shared_v01, v01_loc)
    for c in range(NGROUP):
        sl = pl.ds(c * CHUNK, CHUNK)
        v_loc[sl] = v01_loc[sl] + v01_loc[pl.ds(SPECIES_PAD + c * CHUNK, CHUNK)]

    cp_pk.wait()

    acc_loc[...] = jnp.zeros((N_SYS,), jnp.float32)
    seg0_vec = plsc.bitcast(pk_loc[4, pl.ds(0, CHUNK)], jnp.int32)

    def blk_body(blk, carry):
        vacc, cur = carry
        base = blk * (BLK * CHUNK)
        vals, segs = [], []
        for q in range(BLK):
            sl = pl.ds(base + q * CHUNK, CHUNK)
            x = pk_loc[0, sl]
            y = pk_loc[1, sl]
            z = pk_loc[2, sl]
            a = plsc.bitcast(pk_loc[3, sl], jnp.int32)
            seg = plsc.bitcast(pk_loc[4, sl], jnp.int32)
            rr = x * x + y * y + z * z
            w = plsc.bitcast(_MAGIC - (plsc.bitcast(rr, jnp.int32) >> 1),
                             jnp.float32)
            half = rr * 0.5
            for _ in range(2):
                w = w * (1.5 - half * w * w)
            r = rr * w
            va = plsc.load_gather(v_loc, [a])
            vals.append(r * va)
            segs.append(seg)
        uniform = segs[BLK - 1][CHUNK - 1] == cur

        def fast():
            return vacc + ((vals[0] + vals[1]) + (vals[2] + vals[3])), cur

        def slow():
            plsc.addupdate_scatter(acc_loc, [jnp.full((CHUNK,), cur, jnp.int32)],
                                   vacc)
            for q in range(BLK):
                plsc.addupdate_scatter(acc_loc, [segs[q]], vals[q])
            return jnp.zeros((CHUNK,), jnp.float32), segs[BLK - 1][CHUNK - 1]

        return lax.cond(uniform, fast, slow)

    vacc, cur = lax.fori_loop(
        0, NCHUNK // BLK, blk_body,
        (jnp.zeros((CHUNK,), jnp.float32), seg0_vec[0]))
    plsc.addupdate_scatter(acc_loc, [jnp.full((CHUNK,), cur, jnp.int32)], vacc)

    pltpu.sync_copy(acc_loc, shared_acc.at[pl.ds(wid * N_SYS, N_SYS)])
    plsc.subcore_barrier()

    @pl.when(wid == 0)
    def _():
        pltpu.sync_copy(shared_acc, red_loc)
        pltpu.make_async_copy(bvec_hbm, bvec_loc, sem_b).wait()
        tot = bvec_loc[...]
        for i in range(NTILES):
            tot = tot + red_loc[pl.ds(i * N_SYS, N_SYS)]
        tot_loc[...] = tot
        pltpu.sync_copy(tot_loc, out_hbm)


_sc_kernel = pl.kernel(
    _sc_body,
    out_type=jax.ShapeDtypeStruct((N_SYS,), jnp.float32),
    mesh=plsc.VectorSubcoreMesh(core_axis_name="c", subcore_axis_name="s",
                                num_cores=1, num_subcores=NTILES),
    compiler_params=pltpu.CompilerParams(needs_layout_passes=False),
    scratch_types=[
        pltpu.VMEM((5, APT), jnp.float32),
        pltpu.VMEM((DHALF, CHUNK), jnp.float32),
        pltpu.VMEM((DHALF,), jnp.float32),
        pltpu.VMEM((CHUNK,), jnp.float32),
        pltpu.VMEM((2 * SPECIES_PAD,), jnp.float32),
        pltpu.VMEM((SPECIES_PAD,), jnp.float32),
        pltpu.VMEM((N_SYS,), jnp.float32),
        pltpu.VMEM((NTILES * N_SYS,), jnp.float32),
        pltpu.VMEM((N_SYS,), jnp.float32),
        pltpu.VMEM((N_SYS,), jnp.float32),
        pltpu.VMEM_SHARED((2 * SPECIES_PAD,), jnp.float32),
        pltpu.VMEM_SHARED((NTILES * N_SYS,), jnp.float32),
        pltpu.SemaphoreType.DMA,
        pltpu.SemaphoreType.DMA,
        pltpu.SemaphoreType.DMA,
    ],
)


def kernel(atomic_numbers, pos, batch, species_embed, W, b):
    posf = pos.astype(jnp.float32)
    an_f = lax.bitcast_convert_type(atomic_numbers.astype(jnp.int32), jnp.float32)
    bt_f = lax.bitcast_convert_type(batch.astype(jnp.int32), jnp.float32)
    packed = jnp.stack([posf[:, 0], posf[:, 1], posf[:, 2], an_f, bt_f])
    epad = jnp.pad(species_embed.astype(jnp.float32),
                   ((0, SPECIES_PAD - N_SPECIES), (0, 0)))
    eslab = (epad.reshape(NGROUP, CHUNK, 2, DHALF)
                 .transpose(0, 2, 3, 1)
                 .reshape(2 * NGROUP, DHALF, CHUNK))
    wf = W.reshape(-1).astype(jnp.float32)
    bvec = jnp.broadcast_to(b.astype(jnp.float32), (N_SYS,))
    y = _sc_kernel(packed, eslab, wf, bvec)
    return y.reshape(N_SYS, 1)


# speedup vs baseline: 1.2193x; 1.0646x over previous
import jax
import jax.numpy as jnp
from jax import lax
from jax.experimental import pallas as pl
from jax.experimental.pallas import tpu as pltpu
from jax.experimental.pallas import tpu_sc as plsc

N_ATOMS = 16384
N_SYS = 16
D = 512
N_SPECIES = 100

NTILES = 16
CHUNK = 16
NGROUP = 7
SPECIES_PAD = NGROUP * CHUNK
DHALF = D // 2
APT = N_ATOMS // NTILES
NCHUNK = APT // CHUNK
BLK = 4
EB = 2 * NGROUP * DHALF * CHUNK

_MAGIC = 0x5F3759DF


def _sc_body(pk_hbm, tab_hbm, out_hbm,
             pk_loc, e_loc, w_loc, vchunk, v01_loc, v_loc,
             acc_loc, red_loc, bvec_loc, tot_loc,
             shared_v01, shared_acc, sem_pk, sem_ew, sem_b):
    wid = lax.axis_index("s")

    cp_pk = pltpu.async_copy(pk_hbm.at[:, pl.ds(wid * APT, APT)], pk_loc, sem_pk)

    @pl.when(wid == 0)
    def _():
        pltpu.async_copy(tab_hbm.at[pl.ds(EB + D, N_SYS)], bvec_loc, sem_b)

    @pl.when(wid < 2 * NGROUP)
    def _():
        g = wid // 2
        h = wid % 2
        cp_e = pltpu.async_copy(
            tab_hbm.at[pl.ds(wid * DHALF * CHUNK, DHALF * CHUNK)], e_loc, sem_ew)
        cp_w = pltpu.async_copy(
            tab_hbm.at[pl.ds(EB + h * DHALF, DHALF)], w_loc, sem_ew)
        cp_e.wait()
        cp_w.wait()
        vreg = jnp.zeros((CHUNK,), jnp.float32)
        for d0 in range(0, DHALF, CHUNK):
            wv = w_loc[pl.ds(d0, CHUNK)]
            for j in range(CHUNK):
                vreg = vreg + wv[j] * e_loc[pl.ds((d0 + j) * CHUNK, CHUNK)]
        vchunk[...] = vreg
        pltpu.sync_copy(
            vchunk, shared_v01.at[pl.ds(h * SPECIES_PAD + g * CHUNK, CHUNK)])

    plsc.subcore_barrier()
    pltpu.sync_copy(shared_v01, v01_loc)
    for c in range(NGROUP):
        sl = pl.ds(c * CHUNK, CHUNK)
        v_loc[sl] = v01_loc[sl] + v01_loc[pl.ds(SPECIES_PAD + c * CHUNK, CHUNK)]

    cp_pk.wait()

    acc_loc[...] = jnp.zeros((N_SYS,), jnp.float32)
    seg0_vec = plsc.bitcast(pk_loc[4, pl.ds(0, CHUNK)], jnp.int32)

    def blk_body(blk, carry):
        vacc, cur = carry
        base = blk * (BLK * CHUNK)
        vals, segs = [], []
        for q in range(BLK):
            sl = pl.ds(base + q * CHUNK, CHUNK)
            x = pk_loc[0, sl]
            y = pk_loc[1, sl]
            z = pk_loc[2, sl]
            a = plsc.bitcast(pk_loc[3, sl], jnp.int32)
            seg = plsc.bitcast(pk_loc[4, sl], jnp.int32)
            rr = x * x + y * y + z * z
            w = plsc.bitcast(_MAGIC - (plsc.bitcast(rr, jnp.int32) >> 1),
                             jnp.float32)
            half = rr * 0.5
            for _ in range(2):
                w = w * (1.5 - half * w * w)
            r = rr * w
            va = plsc.load_gather(v_loc, [a])
            vals.append(r * va)
            segs.append(seg)
        uniform = segs[BLK - 1][CHUNK - 1] == cur

        def fast():
            return vacc + ((vals[0] + vals[1]) + (vals[2] + vals[3])), cur

        def slow():
            plsc.addupdate_scatter(acc_loc, [jnp.full((CHUNK,), cur, jnp.int32)],
                                   vacc)
            for q in range(BLK):
                plsc.addupdate_scatter(acc_loc, [segs[q]], vals[q])
            return jnp.zeros((CHUNK,), jnp.float32), segs[BLK - 1][CHUNK - 1]

        return lax.cond(uniform, fast, slow)

    vacc, cur = lax.fori_loop(
        0, NCHUNK // BLK, blk_body,
        (jnp.zeros((CHUNK,), jnp.float32), seg0_vec[0]))
    plsc.addupdate_scatter(acc_loc, [jnp.full((CHUNK,), cur, jnp.int32)], vacc)

    pltpu.sync_copy(acc_loc, shared_acc.at[pl.ds(wid * N_SYS, N_SYS)])
    plsc.subcore_barrier()

    @pl.when(wid == 0)
    def _():
        pltpu.sync_copy(shared_acc, red_loc)
        pltpu.make_async_copy(tab_hbm.at[pl.ds(EB + D, N_SYS)], bvec_loc, sem_b).wait()
        tot = bvec_loc[...]
        for i in range(NTILES):
            tot = tot + red_loc[pl.ds(i * N_SYS, N_SYS)]
        tot_loc[...] = tot
        pltpu.sync_copy(tot_loc, out_hbm)


_sc_kernel = pl.kernel(
    _sc_body,
    out_type=jax.ShapeDtypeStruct((N_SYS,), jnp.float32),
    mesh=plsc.VectorSubcoreMesh(core_axis_name="c", subcore_axis_name="s",
                                num_cores=1, num_subcores=NTILES),
    compiler_params=pltpu.CompilerParams(needs_layout_passes=False),
    scratch_types=[
        pltpu.VMEM((5, APT), jnp.float32),
        pltpu.VMEM((DHALF * CHUNK,), jnp.float32),
        pltpu.VMEM((DHALF,), jnp.float32),
        pltpu.VMEM((CHUNK,), jnp.float32),
        pltpu.VMEM((2 * SPECIES_PAD,), jnp.float32),
        pltpu.VMEM((SPECIES_PAD,), jnp.float32),
        pltpu.VMEM((N_SYS,), jnp.float32),
        pltpu.VMEM((NTILES * N_SYS,), jnp.float32),
        pltpu.VMEM((N_SYS,), jnp.float32),
        pltpu.VMEM((N_SYS,), jnp.float32),
        pltpu.VMEM_SHARED((2 * SPECIES_PAD,), jnp.float32),
        pltpu.VMEM_SHARED((NTILES * N_SYS,), jnp.float32),
        pltpu.SemaphoreType.DMA,
        pltpu.SemaphoreType.DMA,
        pltpu.SemaphoreType.DMA,
    ],
)


def kernel(atomic_numbers, pos, batch, species_embed, W, b):
    posf = pos.astype(jnp.float32)
    an_f = lax.bitcast_convert_type(atomic_numbers.astype(jnp.int32), jnp.float32)
    bt_f = lax.bitcast_convert_type(batch.astype(jnp.int32), jnp.float32)
    packed = jnp.stack([posf[:, 0], posf[:, 1], posf[:, 2], an_f, bt_f])
    epad = jnp.pad(species_embed.astype(jnp.float32),
                   ((0, SPECIES_PAD - N_SPECIES), (0, 0)))
    eslab = (epad.reshape(NGROUP, CHUNK, 2, DHALF)
                 .transpose(0, 2, 3, 1)
                 .reshape(2 * NGROUP, DHALF, CHUNK))
    wf = W.reshape(-1).astype(jnp.float32)
    bvec = jnp.broadcast_to(b.astype(jnp.float32), (N_SYS,))
    tab = jnp.concatenate([eslab.reshape(-1), wf, bvec])
    y = _sc_kernel(packed, tab)
    return y.reshape(N_SYS, 1)


# speedup vs baseline: 1.3926x; 1.1421x over previous
import jax
import jax.numpy as jnp
from jax import lax
from jax.experimental import pallas as pl
from jax.experimental.pallas import tpu as pltpu
from jax.experimental.pallas import tpu_sc as plsc

N_ATOMS = 16384
N_SYS = 16
D = 512
N_SPECIES = 100

NTILES = 16
CHUNK = 16
NGROUP = 7
SPECIES_PAD = NGROUP * CHUNK
DHALF = D // 2
APT = N_ATOMS // NTILES
NCHUNK = APT // CHUNK
BLK = 4
EB = 2 * NGROUP * DHALF * CHUNK
PK = 5 * N_ATOMS

_MAGIC = 0x5F3759DF


def _sc_body(tab_hbm, out_hbm,
             pk_loc, e_loc, w_loc, vchunk, v01_loc, v_loc,
             acc_loc, red_loc, bvec_loc, tot_loc,
             shared_v01, shared_acc, sem_pk, sem_ew, sem_b):
    wid = lax.axis_index("s")

    cps_pk = [
        pltpu.async_copy(tab_hbm.at[pl.ds(q * N_ATOMS + wid * APT, APT)],
                         pk_loc.at[pl.ds(q * APT, APT)], sem_pk)
        for q in range(5)
    ]

    @pl.when(wid == 0)
    def _():
        pltpu.async_copy(tab_hbm.at[pl.ds(PK + EB + D, N_SYS)], bvec_loc, sem_b)

    @pl.when(wid < 2 * NGROUP)
    def _():
        g = wid // 2
        h = wid % 2
        cp_e = pltpu.async_copy(
            tab_hbm.at[pl.ds(PK + wid * DHALF * CHUNK, DHALF * CHUNK)],
            e_loc, sem_ew)
        cp_w = pltpu.async_copy(
            tab_hbm.at[pl.ds(PK + EB + h * DHALF, DHALF)], w_loc, sem_ew)
        cp_e.wait()
        cp_w.wait()
        vreg = jnp.zeros((CHUNK,), jnp.float32)
        for d0 in range(0, DHALF, CHUNK):
            wv = w_loc[pl.ds(d0, CHUNK)]
            for j in range(CHUNK):
                vreg = vreg + wv[j] * e_loc[pl.ds((d0 + j) * CHUNK, CHUNK)]
        vchunk[...] = vreg
        pltpu.sync_copy(
            vchunk, shared_v01.at[pl.ds(h * SPECIES_PAD + g * CHUNK, CHUNK)])

    plsc.subcore_barrier()
    pltpu.sync_copy(shared_v01, v01_loc)
    for c in range(NGROUP):
        sl = pl.ds(c * CHUNK, CHUNK)
        v_loc[sl] = v01_loc[sl] + v01_loc[pl.ds(SPECIES_PAD + c * CHUNK, CHUNK)]

    for cp in cps_pk:
        cp.wait()

    acc_loc[...] = jnp.zeros((N_SYS,), jnp.float32)
    seg0_vec = plsc.bitcast(pk_loc[pl.ds(4 * APT, CHUNK)], jnp.int32)

    def blk_body(blk, carry):
        vacc, cur = carry
        base = blk * (BLK * CHUNK)
        vals, segs = [], []
        for q in range(BLK):
            off = base + q * CHUNK
            x = pk_loc[pl.ds(off, CHUNK)]
            y = pk_loc[pl.ds(APT + off, CHUNK)]
            z = pk_loc[pl.ds(2 * APT + off, CHUNK)]
            a = plsc.bitcast(pk_loc[pl.ds(3 * APT + off, CHUNK)], jnp.int32)
            seg = plsc.bitcast(pk_loc[pl.ds(4 * APT + off, CHUNK)], jnp.int32)
            rr = x * x + y * y + z * z
            w = plsc.bitcast(_MAGIC - (plsc.bitcast(rr, jnp.int32) >> 1),
                             jnp.float32)
            half = rr * 0.5
            for _ in range(2):
                w = w * (1.5 - half * w * w)
            r = rr * w
            va = plsc.load_gather(v_loc, [a])
            vals.append(r * va)
            segs.append(seg)
        uniform = segs[BLK - 1][CHUNK - 1] == cur

        def fast():
            return vacc + ((vals[0] + vals[1]) + (vals[2] + vals[3])), cur

        def slow():
            plsc.addupdate_scatter(acc_loc, [jnp.full((CHUNK,), cur, jnp.int32)],
                                   vacc)
            for q in range(BLK):
                plsc.addupdate_scatter(acc_loc, [segs[q]], vals[q])
            return jnp.zeros((CHUNK,), jnp.float32), segs[BLK - 1][CHUNK - 1]

        return lax.cond(uniform, fast, slow)

    vacc, cur = lax.fori_loop(
        0, NCHUNK // BLK, blk_body,
        (jnp.zeros((CHUNK,), jnp.float32), seg0_vec[0]))
    plsc.addupdate_scatter(acc_loc, [jnp.full((CHUNK,), cur, jnp.int32)], vacc)

    pltpu.sync_copy(acc_loc, shared_acc.at[pl.ds(wid * N_SYS, N_SYS)])
    plsc.subcore_barrier()

    @pl.when(wid == 0)
    def _():
        pltpu.sync_copy(shared_acc, red_loc)
        pltpu.make_async_copy(tab_hbm.at[pl.ds(PK + EB + D, N_SYS)], bvec_loc, sem_b).wait()
        tot = bvec_loc[...]
        for i in range(NTILES):
            tot = tot + red_loc[pl.ds(i * N_SYS, N_SYS)]
        tot_loc[...] = tot
        pltpu.sync_copy(tot_loc, out_hbm)


_sc_kernel = pl.kernel(
    _sc_body,
    out_type=jax.ShapeDtypeStruct((N_SYS,), jnp.float32),
    mesh=plsc.VectorSubcoreMesh(core_axis_name="c", subcore_axis_name="s",
                                num_cores=1, num_subcores=NTILES),
    compiler_params=pltpu.CompilerParams(needs_layout_passes=False),
    scratch_types=[
        pltpu.VMEM((5 * APT,), jnp.float32),
        pltpu.VMEM((DHALF * CHUNK,), jnp.float32),
        pltpu.VMEM((DHALF,), jnp.float32),
        pltpu.VMEM((CHUNK,), jnp.float32),
        pltpu.VMEM((2 * SPECIES_PAD,), jnp.float32),
        pltpu.VMEM((SPECIES_PAD,), jnp.float32),
        pltpu.VMEM((N_SYS,), jnp.float32),
        pltpu.VMEM((NTILES * N_SYS,), jnp.float32),
        pltpu.VMEM((N_SYS,), jnp.float32),
        pltpu.VMEM((N_SYS,), jnp.float32),
        pltpu.VMEM_SHARED((2 * SPECIES_PAD,), jnp.float32),
        pltpu.VMEM_SHARED((NTILES * N_SYS,), jnp.float32),
        pltpu.SemaphoreType.DMA,
        pltpu.SemaphoreType.DMA,
        pltpu.SemaphoreType.DMA,
    ],
)


def kernel(atomic_numbers, pos, batch, species_embed, W, b):
    posf = pos.astype(jnp.float32)
    an_f = lax.bitcast_convert_type(atomic_numbers.astype(jnp.int32), jnp.float32)
    bt_f = lax.bitcast_convert_type(batch.astype(jnp.int32), jnp.float32)
    packed = jnp.stack([posf[:, 0], posf[:, 1], posf[:, 2], an_f, bt_f])
    epad = jnp.pad(species_embed.astype(jnp.float32),
                   ((0, SPECIES_PAD - N_SPECIES), (0, 0)))
    eslab = (epad.reshape(NGROUP, CHUNK, 2, DHALF)
                 .transpose(0, 2, 3, 1)
                 .reshape(2 * NGROUP, DHALF, CHUNK))
    wf = W.reshape(-1).astype(jnp.float32)
    bvec = jnp.broadcast_to(b.astype(jnp.float32), (N_SYS,))
    tab = jnp.concatenate([packed.reshape(-1), eslab.reshape(-1), wf, bvec])
    y = _sc_kernel(tab)
    return y.reshape(N_SYS, 1)
